# baseline (device time: 54924 ns/iter reference)
import jax
import jax.numpy as jnp
from jax import lax
from jax.experimental import pallas as pl
from jax.experimental.pallas import tpu as pltpu

N_DEV = 8

MASKS = ((3, 1, 4), (4, 3, 1), (1, 4, 2))
N_BFLY = 3
N_STAGE = 3
R0 = (0, 384, 768)
BR = (384, 384, 256)


def kernel(x, Wg, Wu, Wd):
    M, D = x.shape
    H = Wg.shape[1]
    Dout = Wd.shape[1]
    HALVES = tuple(tuple(BR[b] // (2 << k) for k in range(N_STAGE))
                   for b in range(N_BFLY))
    SOFF = tuple((0, HALVES[b][0], HALVES[b][0] + HALVES[b][1])
                 for b in range(N_BFLY))
    BUFROWS = max(sum(HALVES[b]) for b in range(N_BFLY))

    def body(x_ref, wg_ref, wu_ref, wd_ref, out_ref,
             sbuf, rbuf, agbuf,
             rs_ssem, rs_rsem, ag_ssem, ag_rsem):
        my = lax.axis_index("i")
        pending_send = []

        barrier = pltpu.get_barrier_semaphore()
        partners = sorted({m for masks in MASKS for m in masks})
        for m in partners:
            pl.semaphore_signal(
                barrier, inc=1, device_id=(my ^ m,),
                device_id_type=pl.DeviceIdType.MESH,
            )
        pl.semaphore_wait(barrier, len(partners))

        def compute_tile(row_base, nrows):
            xs = x_ref[pl.ds(row_base, nrows), :]
            g = jnp.dot(xs, wg_ref[...], preferred_element_type=jnp.float32)
            u = jnp.dot(xs, wu_ref[...], preferred_element_type=jnp.float32)
            h = g * (u * jax.nn.sigmoid(u))
            out_ref[pl.ds(row_base, nrows), :] = jnp.dot(
                h, wd_ref[...], preferred_element_type=jnp.float32)

        def rs_start(b, k, base):
            half = HALVES[b][k]
            p = my ^ MASKS[b][k]
            keep_high = my > p
            send_base = base + jnp.where(keep_high, 0, half)
            sbuf[b, pl.ds(SOFF[b][k], half), :] = (
                out_ref[pl.ds(send_base, half), :].astype(jnp.bfloat16))
            rdma = pltpu.make_async_remote_copy(
                src_ref=sbuf.at[b, pl.ds(SOFF[b][k], half)],
                dst_ref=rbuf.at[b, pl.ds(SOFF[b][k], half)],
                send_sem=rs_ssem.at[b, k],
                recv_sem=rs_rsem.at[b, k],
                device_id=(p,),
                device_id_type=pl.DeviceIdType.MESH,
            )
            rdma.start()
            pending_send.append(rdma)
            return rdma, keep_high

        def rs_finish(b, k, base, rdma, keep_high):
            half = HALVES[b][k]
            rdma.wait_recv()
            my_base = base + jnp.where(keep_high, half, 0)
            rows = pl.ds(my_base, half)
            out_ref[rows, :] = (
                out_ref[rows, :]
                + rbuf[b, pl.ds(SOFF[b][k], half), :].astype(jnp.float32))
            return my_base

        keep0 = [my > (my ^ MASKS[b][0]) for b in range(N_BFLY)]
        send0 = [R0[b] + jnp.where(keep0[b], 0, HALVES[b][0])
                 for b in range(N_BFLY)]
        kept0 = [R0[b] + jnp.where(keep0[b], HALVES[b][0], 0)
                 for b in range(N_BFLY)]

        for b in range(N_BFLY):
            compute_tile(send0[b], HALVES[b][0])
        rdmas = [rs_start(b, 0, jnp.int32(R0[b]))[0] for b in range(N_BFLY)]
        for b in range(N_BFLY):
            compute_tile(kept0[b], HALVES[b][0])

        bases = [None] * N_BFLY
        keeps = [[None] * N_STAGE for _ in range(N_BFLY)]
        for b in range(N_BFLY):
            keeps[b][0] = keep0[b]
            bases[b] = rs_finish(b, 0, jnp.int32(R0[b]), rdmas[b], keep0[b])

        for k in range(1, N_STAGE):
            started = [rs_start(b, k, bases[b]) for b in range(N_BFLY)]
            for b in range(N_BFLY):
                rdma, keep_high = started[b]
                keeps[b][k] = keep_high
                bases[b] = rs_finish(b, k, bases[b], rdma, keep_high)

        for b in range(N_BFLY):
            rows = pl.ds(bases[b], HALVES[b][-1])
            agbuf[rows, :] = out_ref[rows, :].astype(jnp.bfloat16)

        for k in range(N_STAGE - 1, -1, -1):
            rdmas = []
            for b in range(N_BFLY):
                p = my ^ MASKS[b][k]
                rows = pl.ds(bases[b], HALVES[b][k])
                rdma = pltpu.make_async_remote_copy(
                    src_ref=agbuf.at[rows],
                    dst_ref=agbuf.at[rows],
                    send_sem=ag_ssem.at[b, k],
                    recv_sem=ag_rsem.at[b, k],
                    device_id=(p,),
                    device_id_type=pl.DeviceIdType.MESH,
                )
                rdma.start()
                pending_send.append(rdma)
                rdmas.append(rdma)
            if k == 0:
                for b in range(N_BFLY):
                    rows = pl.ds(bases[b], HALVES[b][0])
                    out_ref[rows, :] = agbuf[rows, :].astype(jnp.float32)
            for b in range(N_BFLY):
                half = HALVES[b][k]
                rdmas[b].wait_recv()
                if k == 0:
                    recv_base = bases[b] + jnp.where(keeps[b][k], -half, half)
                    rows = pl.ds(recv_base, half)
                    out_ref[rows, :] = agbuf[rows, :].astype(jnp.float32)
                bases[b] = bases[b] - jnp.where(keeps[b][k], half, 0)

        for rdma in pending_send:
            rdma.wait_send()

    return pl.pallas_call(
        body,
        out_shape=jax.ShapeDtypeStruct((M, Dout), jnp.float32),
        in_specs=[pl.BlockSpec(memory_space=pltpu.VMEM)] * 4,
        out_specs=pl.BlockSpec(memory_space=pltpu.VMEM),
        scratch_shapes=[
            pltpu.VMEM((N_BFLY, BUFROWS, Dout), jnp.bfloat16),
            pltpu.VMEM((N_BFLY, BUFROWS, Dout), jnp.bfloat16),
            pltpu.VMEM((M, Dout), jnp.bfloat16),
            pltpu.SemaphoreType.DMA((N_BFLY, N_STAGE)),
            pltpu.SemaphoreType.DMA((N_BFLY, N_STAGE)),
            pltpu.SemaphoreType.DMA((N_BFLY, N_STAGE)),
            pltpu.SemaphoreType.DMA((N_BFLY, N_STAGE)),
        ],
        compiler_params=pltpu.CompilerParams(collective_id=0),
    )(x, Wg, Wu, Wd)


# device time: 53030 ns/iter; 1.0357x vs baseline; 1.0357x over previous
import jax
import jax.numpy as jnp
from jax import lax
from jax.experimental import pallas as pl
from jax.experimental.pallas import tpu as pltpu

N_DEV = 8

MASKS = ((3, 1, 4), (4, 3, 1), (1, 4, 2))
N_BFLY = 3
N_STAGE = 3
R0 = (0, 384, 768)
BR = (384, 384, 256)


def kernel(x, Wg, Wu, Wd):
    M, D = x.shape
    H = Wg.shape[1]
    Dout = Wd.shape[1]
    HALVES = tuple(tuple(BR[b] // (2 << k) for k in range(N_STAGE))
                   for b in range(N_BFLY))
    SOFF = tuple((0, HALVES[b][0], HALVES[b][0] + HALVES[b][1])
                 for b in range(N_BFLY))
    BUFROWS = max(SOFF[b][2] + HALVES[b][1] for b in range(N_BFLY))

    def body(x_ref, wg_ref, wu_ref, wd_ref, out_ref,
             sbuf, rbuf, agbuf,
             rs_ssem, rs_rsem, ag_ssem, ag_rsem):
        my = lax.axis_index("i")
        pending_send = []

        barrier = pltpu.get_barrier_semaphore()
        partners = sorted({m for masks in MASKS for m in masks})
        for m in partners:
            pl.semaphore_signal(
                barrier, inc=1, device_id=(my ^ m,),
                device_id_type=pl.DeviceIdType.MESH,
            )
        pl.semaphore_wait(barrier, len(partners))

        def compute_tile(row_base, nrows):
            xs = x_ref[pl.ds(row_base, nrows), :]
            g = jnp.dot(xs, wg_ref[...], preferred_element_type=jnp.float32)
            u = jnp.dot(xs, wu_ref[...], preferred_element_type=jnp.float32)
            h = g * (u * jax.nn.sigmoid(u))
            out_ref[pl.ds(row_base, nrows), :] = jnp.dot(
                h, wd_ref[...], preferred_element_type=jnp.float32)

        def rs_start(b, k, base):
            half = HALVES[b][k]
            p = my ^ MASKS[b][k]
            keep_high = my > p
            send_base = base + jnp.where(keep_high, 0, half)
            sbuf[b, pl.ds(SOFF[b][k], half), :] = (
                out_ref[pl.ds(send_base, half), :].astype(jnp.bfloat16))
            rdma = pltpu.make_async_remote_copy(
                src_ref=sbuf.at[b, pl.ds(SOFF[b][k], half)],
                dst_ref=rbuf.at[b, pl.ds(SOFF[b][k], half)],
                send_sem=rs_ssem.at[b, k],
                recv_sem=rs_rsem.at[b, k],
                device_id=(p,),
                device_id_type=pl.DeviceIdType.MESH,
            )
            rdma.start()
            pending_send.append(rdma)
            return rdma, keep_high

        def rs_finish(b, k, base, rdma, keep_high):
            half = HALVES[b][k]
            rdma.wait_recv()
            my_base = base + jnp.where(keep_high, half, 0)
            rows = pl.ds(my_base, half)
            out_ref[rows, :] = (
                out_ref[rows, :]
                + rbuf[b, pl.ds(SOFF[b][k], half), :].astype(jnp.float32))
            return my_base

        keep0 = [my > (my ^ MASKS[b][0]) for b in range(N_BFLY)]
        send0 = [R0[b] + jnp.where(keep0[b], 0, HALVES[b][0])
                 for b in range(N_BFLY)]
        kept0 = [R0[b] + jnp.where(keep0[b], HALVES[b][0], 0)
                 for b in range(N_BFLY)]

        rdmas = []
        for b in range(N_BFLY):
            compute_tile(send0[b], HALVES[b][0])
            rdmas.append(rs_start(b, 0, jnp.int32(R0[b]))[0])
        for b in range(N_BFLY):
            compute_tile(kept0[b], HALVES[b][0])

        bases = [None] * N_BFLY
        keeps = [[None] * N_STAGE for _ in range(N_BFLY)]
        for b in range(N_BFLY):
            keeps[b][0] = keep0[b]
            bases[b] = rs_finish(b, 0, jnp.int32(R0[b]), rdmas[b], keep0[b])

        started = [rs_start(b, 1, bases[b]) for b in range(N_BFLY)]
        for b in range(N_BFLY):
            rdma, keep_high = started[b]
            keeps[b][1] = keep_high
            bases[b] = rs_finish(b, 1, bases[b], rdma, keep_high)

        started = []
        for b in range(N_BFLY):
            seg = HALVES[b][1]
            p = my ^ MASKS[b][2]
            sbuf[b, pl.ds(SOFF[b][2], seg), :] = (
                out_ref[pl.ds(bases[b], seg), :].astype(jnp.bfloat16))
            rdma = pltpu.make_async_remote_copy(
                src_ref=sbuf.at[b, pl.ds(SOFF[b][2], seg)],
                dst_ref=rbuf.at[b, pl.ds(SOFF[b][2], seg)],
                send_sem=rs_ssem.at[b, 2],
                recv_sem=rs_rsem.at[b, 2],
                device_id=(p,),
                device_id_type=pl.DeviceIdType.MESH,
            )
            rdma.start()
            pending_send.append(rdma)
            started.append(rdma)
        for b in range(N_BFLY):
            seg = HALVES[b][1]
            started[b].wait_recv()
            rows = pl.ds(bases[b], seg)
            out_ref[rows, :] = (
                out_ref[rows, :]
                + rbuf[b, pl.ds(SOFF[b][2], seg), :].astype(jnp.float32))
            agbuf[rows, :] = out_ref[rows, :].astype(jnp.bfloat16)

        for k in range(N_STAGE - 2, -1, -1):
            rdmas = []
            for b in range(N_BFLY):
                p = my ^ MASKS[b][k]
                rows = pl.ds(bases[b], HALVES[b][k])
                rdma = pltpu.make_async_remote_copy(
                    src_ref=agbuf.at[rows],
                    dst_ref=agbuf.at[rows],
                    send_sem=ag_ssem.at[b, k],
                    recv_sem=ag_rsem.at[b, k],
                    device_id=(p,),
                    device_id_type=pl.DeviceIdType.MESH,
                )
                rdma.start()
                pending_send.append(rdma)
                rdmas.append(rdma)
            if k == 0:
                for b in range(N_BFLY):
                    rows = pl.ds(bases[b], HALVES[b][0])
                    out_ref[rows, :] = agbuf[rows, :].astype(jnp.float32)
            for b in range(N_BFLY):
                half = HALVES[b][k]
                rdmas[b].wait_recv()
                if k == 0:
                    recv_base = bases[b] + jnp.where(keeps[b][k], -half, half)
                    rows = pl.ds(recv_base, half)
                    out_ref[rows, :] = agbuf[rows, :].astype(jnp.float32)
                bases[b] = bases[b] - jnp.where(keeps[b][k], half, 0)

        for rdma in pending_send:
            rdma.wait_send()

    return pl.pallas_call(
        body,
        out_shape=jax.ShapeDtypeStruct((M, Dout), jnp.float32),
        in_specs=[pl.BlockSpec(memory_space=pltpu.VMEM)] * 4,
        out_specs=pl.BlockSpec(memory_space=pltpu.VMEM),
        scratch_shapes=[
            pltpu.VMEM((N_BFLY, BUFROWS, Dout), jnp.bfloat16),
            pltpu.VMEM((N_BFLY, BUFROWS, Dout), jnp.bfloat16),
            pltpu.VMEM((M, Dout), jnp.bfloat16),
            pltpu.SemaphoreType.DMA((N_BFLY, N_STAGE)),
            pltpu.SemaphoreType.DMA((N_BFLY, N_STAGE)),
            pltpu.SemaphoreType.DMA((N_BFLY, N_STAGE)),
            pltpu.SemaphoreType.DMA((N_BFLY, N_STAGE)),
        ],
        compiler_params=pltpu.CompilerParams(collective_id=0),
    )(x, Wg, Wu, Wd)
